# baseline (speedup 1.0000x reference)
import jax
import jax.numpy as jnp
from jax.experimental import pallas as pl
from jax.experimental.pallas import tpu as pltpu

_H = 512
_W = 512
_STRIP = 16
_IPB = 8


def _erode_pair(xs, prev_row):
    xu = jnp.concatenate([prev_row, xs[:-1]], axis=0)
    r = jnp.minimum(xs, xu)
    rm = jnp.concatenate([r[:, :1], r[:, :-1]], axis=1)
    rp = jnp.concatenate([r[:, 1:], r[:, _W - 2:_W - 1]], axis=1)
    return jnp.minimum(r, rm), jnp.minimum(r, rp)


def _dilate_sqerr(g, g_row, x):
    gd = jnp.concatenate([g[1:], g_row], axis=0)
    opened = jnp.maximum(g, gd)
    diff = x - opened
    return diff * diff


def _fold(acc, d2):
    for m in range(d2.shape[0] // 8):
        acc = acc + d2[8 * m:8 * m + 8]
    return acc


def _opening_mse_body(x_ref, out_ref):
    j = pl.program_id(1)
    n_strips = _H // _STRIP

    unroll = 6

    def img_body(k, acc):
        xs0 = x_ref[k, 0:_STRIP, :]
        e0, en0 = _erode_pair(xs0, xs0[0:1])
        g0 = jnp.maximum(e0, en0)

        def group_body(gi, carry):
            x_prev, g_prev, a = carry
            base = _STRIP + gi * (unroll * _STRIP)
            for u in range(unroll):
                r0 = pl.multiple_of(base + u * _STRIP, _STRIP)
                xs = x_ref[k, pl.ds(r0, _STRIP), :]
                e, en = _erode_pair(xs, x_prev[_STRIP - 1:_STRIP])
                g = jnp.maximum(e, en)
                a = _fold(a, _dilate_sqerr(g_prev, g[0:1], x_prev))
                x_prev, g_prev = xs, g
            return (x_prev, g_prev, a)

        n_groups = (n_strips - 2) // unroll
        x_prev, g_prev, acc = jax.lax.fori_loop(
            0, n_groups, group_body, (xs0, g0, acc))
        s = n_strips - 1
        xs = x_ref[k, s * _STRIP:(s + 1) * _STRIP, :]
        e, en = _erode_pair(xs, x_prev[_STRIP - 1:_STRIP])
        g = jnp.maximum(e, en)
        acc = _fold(acc, _dilate_sqerr(g_prev, g[0:1], x_prev))
        last = _STRIP - 1
        return _fold(acc, _dilate_sqerr(g, g[last:last + 1], xs))

    acc = jax.lax.fori_loop(
        0, _IPB, img_body, jnp.zeros((8, _W), jnp.float32))
    total = jnp.sum(acc).reshape(1, 1, 1)

    @pl.when(j == 0)
    def _():
        out_ref[...] = total

    @pl.when(j != 0)
    def _():
        out_ref[...] = out_ref[...] + total


def kernel(labels):
    b, c, h, w = labels.shape
    n = b * c
    x = labels.reshape(n, h, w)
    per_core = n // 2 // _IPB
    partials = pl.pallas_call(
        _opening_mse_body,
        grid=(2, per_core),
        in_specs=[pl.BlockSpec((_IPB, h, w),
                               lambda i, j: (i * per_core + j, 0, 0))],
        out_specs=pl.BlockSpec((1, 1, 1), lambda i, j: (i, 0, 0)),
        out_shape=jax.ShapeDtypeStruct((2, 1, 1), jnp.float32),
        compiler_params=pltpu.CompilerParams(
            dimension_semantics=("parallel", "arbitrary"),
        ),
    )(x)
    return jnp.sum(partials) / (n * h * w)


# speedup vs baseline: 1.5758x; 1.5758x over previous
import jax
import jax.numpy as jnp
from jax.experimental import pallas as pl
from jax.experimental.pallas import tpu as pltpu

_H = 512
_W = 512
_STRIP = 16
_IPB = 8


def _lane_dilated_erosion(xs, prev_row):
    xu = jnp.concatenate([prev_row, xs[:-1]], axis=0)
    r = jnp.minimum(xs, xu)
    rm = jnp.concatenate([r[:, :1], r[:, :-1]], axis=1)
    rp = jnp.concatenate([r[:, 1:], r[:, _W - 2:_W - 1]], axis=1)
    return jnp.minimum(r, jnp.maximum(rm, rp))


def _dilate_sqerr(g, g_row, x):
    gd = jnp.concatenate([g[1:], g_row], axis=0)
    opened = jnp.maximum(g, gd)
    diff = x - opened
    return diff * diff


def _fold(acc, d2):
    for m in range(d2.shape[0] // 8):
        acc = acc + d2[8 * m:8 * m + 8]
    return acc


def _opening_mse_body(x_ref, out_ref):
    j = pl.program_id(0)
    n_strips = _H // _STRIP

    def img_body(k, acc):
        x_prev = g_prev = None
        for s in range(n_strips):
            xs = x_ref[k, s * _STRIP:(s + 1) * _STRIP, :]
            if s == 0:
                prev_row = xs[0:1]
            else:
                prev_row = x_prev[_STRIP - 1:_STRIP]
            g = _lane_dilated_erosion(xs, prev_row)
            if s > 0:
                acc = _fold(acc, _dilate_sqerr(g_prev, g[0:1], x_prev))
            x_prev, g_prev = xs, g
        last = _STRIP - 1
        return _fold(acc, _dilate_sqerr(
            g_prev, g_prev[last:last + 1], x_prev))

    acc = jax.lax.fori_loop(
        0, _IPB, img_body, jnp.zeros((8, _W), jnp.float32))
    total = jnp.sum(acc).reshape(1, 1, 1)

    @pl.when(j == 0)
    def _():
        out_ref[...] = total

    @pl.when(j != 0)
    def _():
        out_ref[...] = out_ref[...] + total


def kernel(labels):
    b, c, h, w = labels.shape
    n = b * c
    x = labels.reshape(n, h, w)
    steps = n // _IPB
    partials = pl.pallas_call(
        _opening_mse_body,
        grid=(steps,),
        in_specs=[pl.BlockSpec((_IPB, h, w), lambda j: (j, 0, 0))],
        out_specs=pl.BlockSpec((1, 1, 1), lambda j: (0, 0, 0)),
        out_shape=jax.ShapeDtypeStruct((1, 1, 1), jnp.float32),
        compiler_params=pltpu.CompilerParams(
            dimension_semantics=("arbitrary",),
        ),
    )(x)
    return jnp.sum(partials) / (n * h * w)


# speedup vs baseline: 1.6057x; 1.0190x over previous
import jax
import jax.numpy as jnp
from jax.experimental import pallas as pl
from jax.experimental.pallas import tpu as pltpu

_H = 512
_W = 512
_STRIP = 16
_IPB = 8


def _lane_dilated_erosion(xs, xu):
    r = jnp.minimum(xs, xu)
    rm = jnp.concatenate([r[:, :1], r[:, :-1]], axis=1)
    rp = jnp.concatenate([r[:, 1:], r[:, _W - 2:_W - 1]], axis=1)
    return jnp.minimum(r, jnp.maximum(rm, rp))


def _dilate_sqerr(g, g_row, x):
    gd = jnp.concatenate([g[1:], g_row], axis=0)
    opened = jnp.maximum(g, gd)
    diff = x - opened
    return diff * diff


def _fold(acc, d2):
    for m in range(d2.shape[0] // 8):
        acc = acc + d2[8 * m:8 * m + 8]
    return acc


def _opening_mse_body(x_ref, out_ref):
    j = pl.program_id(0)
    n_strips = _H // _STRIP

    def img_body(k, acc):
        g_prev = None
        for s in range(n_strips):
            r0 = s * _STRIP
            xs = x_ref[k, r0:r0 + _STRIP, :]
            if s == 0:
                xu = jnp.concatenate([xs[0:1], xs[:-1]], axis=0)
            else:
                xu = x_ref[k, r0 - 1:r0 + _STRIP - 1, :]
            g = _lane_dilated_erosion(xs, xu)
            if s > 0:
                xp = x_ref[k, r0 - _STRIP:r0, :]
                acc = _fold(acc, _dilate_sqerr(g_prev, g[0:1], xp))
            g_prev = g
        last = _STRIP - 1
        xp = x_ref[k, _H - _STRIP:_H, :]
        return _fold(acc, _dilate_sqerr(
            g_prev, g_prev[last:last + 1], xp))

    acc = jax.lax.fori_loop(
        0, _IPB, img_body, jnp.zeros((8, _W), jnp.float32))
    total = jnp.sum(acc).reshape(1, 1, 1)

    @pl.when(j == 0)
    def _():
        out_ref[...] = total

    @pl.when(j != 0)
    def _():
        out_ref[...] = out_ref[...] + total


def kernel(labels):
    b, c, h, w = labels.shape
    n = b * c
    x = labels.reshape(n, h, w)
    steps = n // _IPB
    partials = pl.pallas_call(
        _opening_mse_body,
        grid=(steps,),
        in_specs=[pl.BlockSpec((_IPB, h, w), lambda j: (j, 0, 0))],
        out_specs=pl.BlockSpec((1, 1, 1), lambda j: (0, 0, 0)),
        out_shape=jax.ShapeDtypeStruct((1, 1, 1), jnp.float32),
        compiler_params=pltpu.CompilerParams(
            dimension_semantics=("arbitrary",),
        ),
    )(x)
    return jnp.sum(partials) / (n * h * w)


# speedup vs baseline: 1.6095x; 1.0024x over previous
import jax
import jax.numpy as jnp
from jax.experimental import pallas as pl
from jax.experimental.pallas import tpu as pltpu

_H = 512
_W = 512
_STRIP = 32
_IPB = 8


def _lane_dilated_erosion(xs, xu):
    r = jnp.minimum(xs, xu)
    rm = jnp.concatenate([r[:, :1], r[:, :-1]], axis=1)
    rp = jnp.concatenate([r[:, 1:], r[:, _W - 2:_W - 1]], axis=1)
    return jnp.minimum(r, jnp.maximum(rm, rp))


def _dilate_sqerr(g, g_row, x):
    gd = jnp.concatenate([g[1:], g_row], axis=0)
    opened = jnp.maximum(g, gd)
    diff = x - opened
    return diff * diff


def _fold(acc, d2):
    for m in range(d2.shape[0] // 8):
        acc = acc + d2[8 * m:8 * m + 8]
    return acc


def _opening_mse_body(x_ref, out_ref):
    j = pl.program_id(0)
    n_strips = _H // _STRIP

    def img_body(k, acc):
        g_prev = None
        for s in range(n_strips):
            r0 = s * _STRIP
            xs = x_ref[k, r0:r0 + _STRIP, :]
            if s == 0:
                xu = jnp.concatenate([xs[0:1], xs[:-1]], axis=0)
            else:
                xu = x_ref[k, r0 - 1:r0 + _STRIP - 1, :]
            g = _lane_dilated_erosion(xs, xu)
            if s > 0:
                xp = x_ref[k, r0 - _STRIP:r0, :]
                acc = _fold(acc, _dilate_sqerr(g_prev, g[0:1], xp))
            g_prev = g
        last = _STRIP - 1
        xp = x_ref[k, _H - _STRIP:_H, :]
        return _fold(acc, _dilate_sqerr(
            g_prev, g_prev[last:last + 1], xp))

    acc = jax.lax.fori_loop(
        0, _IPB, img_body, jnp.zeros((8, _W), jnp.float32))
    total = jnp.sum(acc).reshape(1, 1, 1)

    @pl.when(j == 0)
    def _():
        out_ref[...] = total

    @pl.when(j != 0)
    def _():
        out_ref[...] = out_ref[...] + total


def kernel(labels):
    b, c, h, w = labels.shape
    n = b * c
    x = labels.reshape(n, h, w)
    steps = n // _IPB
    partials = pl.pallas_call(
        _opening_mse_body,
        grid=(steps,),
        in_specs=[pl.BlockSpec((_IPB, h, w), lambda j: (j, 0, 0))],
        out_specs=pl.BlockSpec((1, 1, 1), lambda j: (0, 0, 0)),
        out_shape=jax.ShapeDtypeStruct((1, 1, 1), jnp.float32),
        compiler_params=pltpu.CompilerParams(
            dimension_semantics=("arbitrary",),
        ),
    )(x)
    return jnp.sum(partials) / (n * h * w)


# speedup vs baseline: 1.6645x; 1.0342x over previous
import jax
import jax.numpy as jnp
from jax.experimental import pallas as pl
from jax.experimental.pallas import tpu as pltpu

_H = 512
_W = 512
_STRIP = 16
_IPB = 8


def _lane_dilated_erosion(xs, xu):
    r = jnp.minimum(xs, xu)
    rm = jnp.concatenate([r[:, :1], r[:, :-1]], axis=1)
    rp = jnp.concatenate([r[:, 1:], r[:, _W - 2:_W - 1]], axis=1)
    return jnp.minimum(r, jnp.maximum(rm, rp))


def _dilate_sqerr(g, g_row, x):
    gd = jnp.concatenate([g[1:], g_row], axis=0)
    opened = jnp.maximum(g, gd)
    diff = x - opened
    return diff * diff


def _fold(acc, d2):
    for m in range(d2.shape[0] // 8):
        acc = acc + d2[8 * m:8 * m + 8]
    return acc


def _opening_mse_body(x_ref, out_ref):
    j = pl.program_id(0)
    n_strips = _H // _STRIP

    def pair_body(p, acc):
        ka = 2 * p
        kb = 2 * p + 1
        g_prev = [None, None]
        last = _STRIP - 1
        for s in range(n_strips):
            r0 = s * _STRIP
            for i, k in ((0, ka), (1, kb)):
                xs = x_ref[k, r0:r0 + _STRIP, :]
                if s == 0:
                    xu = jnp.concatenate([xs[0:1], xs[:-1]], axis=0)
                else:
                    xu = x_ref[k, r0 - 1:r0 + _STRIP - 1, :]
                g = _lane_dilated_erosion(xs, xu)
                if s > 0:
                    xp = x_ref[k, r0 - _STRIP:r0, :]
                    acc = _fold(acc, _dilate_sqerr(g_prev[i], g[0:1], xp))
                g_prev[i] = g
        for i, k in ((0, ka), (1, kb)):
            xp = x_ref[k, _H - _STRIP:_H, :]
            acc = _fold(acc, _dilate_sqerr(
                g_prev[i], g_prev[i][last:last + 1], xp))
        return acc

    acc = jax.lax.fori_loop(
        0, _IPB // 2, pair_body, jnp.zeros((8, _W), jnp.float32))
    total = jnp.sum(acc).reshape(1, 1, 1)

    @pl.when(j == 0)
    def _():
        out_ref[...] = total

    @pl.when(j != 0)
    def _():
        out_ref[...] = out_ref[...] + total


def kernel(labels):
    b, c, h, w = labels.shape
    n = b * c
    x = labels.reshape(n, h, w)
    steps = n // _IPB
    partials = pl.pallas_call(
        _opening_mse_body,
        grid=(steps,),
        in_specs=[pl.BlockSpec((_IPB, h, w), lambda j: (j, 0, 0))],
        out_specs=pl.BlockSpec((1, 1, 1), lambda j: (0, 0, 0)),
        out_shape=jax.ShapeDtypeStruct((1, 1, 1), jnp.float32),
        compiler_params=pltpu.CompilerParams(
            dimension_semantics=("arbitrary",),
        ),
    )(x)
    return jnp.sum(partials) / (n * h * w)


# speedup vs baseline: 1.7002x; 1.0214x over previous
import jax
import jax.numpy as jnp
from jax.experimental import pallas as pl
from jax.experimental.pallas import tpu as pltpu

_H = 512
_W = 512
_STRIP = 8
_IPB = 8


def _lane_dilated_erosion(xs, xu):
    r = jnp.minimum(xs, xu)
    rm = jnp.concatenate([r[:, :1], r[:, :-1]], axis=1)
    rp = jnp.concatenate([r[:, 1:], r[:, _W - 2:_W - 1]], axis=1)
    return jnp.minimum(r, jnp.maximum(rm, rp))


def _dilate_sqerr(g, g_row, x):
    gd = jnp.concatenate([g[1:], g_row], axis=0)
    opened = jnp.maximum(g, gd)
    diff = x - opened
    return diff * diff


def _fold(acc, d2):
    for m in range(d2.shape[0] // 8):
        acc = acc + d2[8 * m:8 * m + 8]
    return acc


def _opening_mse_body(x_ref, out_ref):
    j = pl.program_id(0)
    n_strips = _H // _STRIP

    def pair_body(p, acc):
        ka = 2 * p
        kb = 2 * p + 1
        g_prev = [None, None]
        last = _STRIP - 1
        for s in range(n_strips):
            r0 = s * _STRIP
            for i, k in ((0, ka), (1, kb)):
                xs = x_ref[k, r0:r0 + _STRIP, :]
                if s == 0:
                    xu = jnp.concatenate([xs[0:1], xs[:-1]], axis=0)
                else:
                    xu = x_ref[k, r0 - 1:r0 + _STRIP - 1, :]
                g = _lane_dilated_erosion(xs, xu)
                if s > 0:
                    xp = x_ref[k, r0 - _STRIP:r0, :]
                    acc = _fold(acc, _dilate_sqerr(g_prev[i], g[0:1], xp))
                g_prev[i] = g
        for i, k in ((0, ka), (1, kb)):
            xp = x_ref[k, _H - _STRIP:_H, :]
            acc = _fold(acc, _dilate_sqerr(
                g_prev[i], g_prev[i][last:last + 1], xp))
        return acc

    acc = jax.lax.fori_loop(
        0, _IPB // 2, pair_body, jnp.zeros((8, _W), jnp.float32))
    total = jnp.sum(acc).reshape(1, 1, 1)

    @pl.when(j == 0)
    def _():
        out_ref[...] = total

    @pl.when(j != 0)
    def _():
        out_ref[...] = out_ref[...] + total


def kernel(labels):
    b, c, h, w = labels.shape
    n = b * c
    x = labels.reshape(n, h, w)
    steps = n // _IPB
    partials = pl.pallas_call(
        _opening_mse_body,
        grid=(steps,),
        in_specs=[pl.BlockSpec((_IPB, h, w), lambda j: (j, 0, 0))],
        out_specs=pl.BlockSpec((1, 1, 1), lambda j: (0, 0, 0)),
        out_shape=jax.ShapeDtypeStruct((1, 1, 1), jnp.float32),
        compiler_params=pltpu.CompilerParams(
            dimension_semantics=("arbitrary",),
        ),
    )(x)
    return jnp.sum(partials) / (n * h * w)


# speedup vs baseline: 1.7338x; 1.0198x over previous
import jax
import jax.numpy as jnp
from jax.experimental import pallas as pl
from jax.experimental.pallas import tpu as pltpu

_H = 512
_W = 512
_STRIP = 8
_IPB = 8


def _lane_dilated_erosion(xs, xu):
    r = jnp.minimum(xs, xu)
    rm = jnp.concatenate([r[:, :1], r[:, :-1]], axis=1)
    rp = jnp.concatenate([r[:, 1:], r[:, _W - 2:_W - 1]], axis=1)
    return jnp.minimum(r, jnp.maximum(rm, rp))


def _dilate_sqerr(g, g_row, x):
    gd = jnp.concatenate([g[1:], g_row], axis=0)
    opened = jnp.maximum(g, gd)
    diff = x - opened
    return diff * diff


def _fold(acc, d2):
    for m in range(d2.shape[0] // 8):
        acc = acc + d2[8 * m:8 * m + 8]
    return acc


def _opening_mse_body(x_ref, out_ref):
    j = pl.program_id(0)
    n_strips = _H // _STRIP

    group = 4

    def pair_body(p, acc):
        ks = [group * p + i for i in range(group)]
        g_prev = [None] * group
        last = _STRIP - 1
        for s in range(n_strips):
            r0 = s * _STRIP
            for i, k in enumerate(ks):
                xs = x_ref[k, r0:r0 + _STRIP, :]
                if s == 0:
                    xu = jnp.concatenate([xs[0:1], xs[:-1]], axis=0)
                else:
                    xu = x_ref[k, r0 - 1:r0 + _STRIP - 1, :]
                g = _lane_dilated_erosion(xs, xu)
                if s > 0:
                    xp = x_ref[k, r0 - _STRIP:r0, :]
                    acc = _fold(acc, _dilate_sqerr(g_prev[i], g[0:1], xp))
                g_prev[i] = g
        for i, k in enumerate(ks):
            xp = x_ref[k, _H - _STRIP:_H, :]
            acc = _fold(acc, _dilate_sqerr(
                g_prev[i], g_prev[i][last:last + 1], xp))
        return acc

    acc = jax.lax.fori_loop(
        0, _IPB // group, pair_body, jnp.zeros((8, _W), jnp.float32))
    total = jnp.sum(acc).reshape(1, 1, 1)

    @pl.when(j == 0)
    def _():
        out_ref[...] = total

    @pl.when(j != 0)
    def _():
        out_ref[...] = out_ref[...] + total


def kernel(labels):
    b, c, h, w = labels.shape
    n = b * c
    x = labels.reshape(n, h, w)
    steps = n // _IPB
    partials = pl.pallas_call(
        _opening_mse_body,
        grid=(steps,),
        in_specs=[pl.BlockSpec((_IPB, h, w), lambda j: (j, 0, 0))],
        out_specs=pl.BlockSpec((1, 1, 1), lambda j: (0, 0, 0)),
        out_shape=jax.ShapeDtypeStruct((1, 1, 1), jnp.float32),
        compiler_params=pltpu.CompilerParams(
            dimension_semantics=("arbitrary",),
        ),
    )(x)
    return jnp.sum(partials) / (n * h * w)


# speedup vs baseline: 1.7446x; 1.0062x over previous
import jax
import jax.numpy as jnp
from jax.experimental import pallas as pl
from jax.experimental.pallas import tpu as pltpu

_H = 512
_W = 512
_STRIP = 8
_IPB = 8


def _lane_dilated_erosion(xs, xu):
    r = jnp.minimum(xs, xu)
    rm = jnp.concatenate([r[:, :1], r[:, :-1]], axis=1)
    rp = jnp.concatenate([r[:, 1:], r[:, _W - 2:_W - 1]], axis=1)
    return jnp.minimum(r, jnp.maximum(rm, rp))


def _dilate_sqerr(g, g_row, x):
    gd = jnp.concatenate([g[1:], g_row], axis=0)
    opened = jnp.maximum(g, gd)
    diff = x - opened
    return diff * diff


def _fold(acc, d2):
    for m in range(d2.shape[0] // 8):
        acc = acc + d2[8 * m:8 * m + 8]
    return acc


def _opening_mse_body(x_ref, out_ref):
    j = pl.program_id(0)
    n_strips = _H // _STRIP

    group = 8

    def pair_body(p, acc):
        ks = [group * p + i for i in range(group)]
        g_prev = [None] * group
        last = _STRIP - 1
        for s in range(n_strips):
            r0 = s * _STRIP
            for i, k in enumerate(ks):
                xs = x_ref[k, r0:r0 + _STRIP, :]
                if s == 0:
                    xu = jnp.concatenate([xs[0:1], xs[:-1]], axis=0)
                else:
                    xu = x_ref[k, r0 - 1:r0 + _STRIP - 1, :]
                g = _lane_dilated_erosion(xs, xu)
                if s > 0:
                    xp = x_ref[k, r0 - _STRIP:r0, :]
                    acc = _fold(acc, _dilate_sqerr(g_prev[i], g[0:1], xp))
                g_prev[i] = g
        for i, k in enumerate(ks):
            xp = x_ref[k, _H - _STRIP:_H, :]
            acc = _fold(acc, _dilate_sqerr(
                g_prev[i], g_prev[i][last:last + 1], xp))
        return acc

    acc = jax.lax.fori_loop(
        0, _IPB // group, pair_body, jnp.zeros((8, _W), jnp.float32))
    total = jnp.sum(acc).reshape(1, 1, 1)

    @pl.when(j == 0)
    def _():
        out_ref[...] = total

    @pl.when(j != 0)
    def _():
        out_ref[...] = out_ref[...] + total


def kernel(labels):
    b, c, h, w = labels.shape
    n = b * c
    x = labels.reshape(n, h, w)
    steps = n // _IPB
    partials = pl.pallas_call(
        _opening_mse_body,
        grid=(steps,),
        in_specs=[pl.BlockSpec((_IPB, h, w), lambda j: (j, 0, 0))],
        out_specs=pl.BlockSpec((1, 1, 1), lambda j: (0, 0, 0)),
        out_shape=jax.ShapeDtypeStruct((1, 1, 1), jnp.float32),
        compiler_params=pltpu.CompilerParams(
            dimension_semantics=("arbitrary",),
        ),
    )(x)
    return jnp.sum(partials) / (n * h * w)
